# baseline (speedup 1.0000x reference)
import math

import jax
import jax.numpy as jnp
import numpy as np
from jax.experimental import pallas as pl
from jax.experimental.pallas import tpu as pltpu


_ROTS = ((13, 15, 26, 6), (17, 29, 16, 24))


def _np_threefry2x32(k0, k1, x0, x1):
    k0 = np.uint32(k0)
    k1 = np.uint32(k1)
    x0 = np.asarray(x0, np.uint32)
    x1 = np.asarray(x1, np.uint32)
    ks = [k0, k1, np.uint32(k0 ^ k1 ^ np.uint32(0x1BD11BDA))]
    x0 = x0 + ks[0]
    x1 = x1 + ks[1]
    for i in range(5):
        for r in _ROTS[i % 2]:
            x0 = x0 + x1
            x1 = (x1 << np.uint32(r)) | (x1 >> np.uint32(32 - r))
            x1 = x1 ^ x0
        x0 = x0 + ks[(i + 1) % 3]
        x1 = x1 + ks[(i + 2) % 3] + np.uint32(i + 1)
    return x0, x1


def _subkeys():
    cnt = np.arange(6, dtype=np.uint32)
    o0, o1 = _np_threefry2x32(0, 42, np.zeros(6, np.uint32), cnt)
    return np.stack([o0, o1], axis=1)


_SK = _subkeys()
_KB, _KC, _KG, _KA, _KB2, _KCON = (tuple(int(v) for v in row) for row in _SK)

_TINY = np.float32(np.finfo(np.float32).tiny)
_NLO = np.float32(np.nextafter(np.float32(-1.0), np.float32(0.0)))
_NSPAN = np.float32(np.float32(1.0) - _NLO)
_SQRT2 = np.float32(np.sqrt(2.0))
_LOGEPS = np.float32(1e-30)


def _schedule():
    steps = 1001
    t = jnp.linspace(0.0, 1.0, steps)
    cum_prec = jnp.cos((t + 0.008) * 0.5 * math.pi / (1 + 0.008)) ** 2 * 1.00015543316
    cum_var = 1.0 - cum_prec
    sqrt_cum_prec = jnp.sqrt(cum_prec)
    sqrt_cum_var = jnp.sqrt(jnp.maximum(cum_var, 0.0))
    return sqrt_cum_prec, sqrt_cum_var


def _tf2x32(k0, k1, ks2, x1_in):
    x0 = k0
    x1 = x1_in + k1
    ks = (k0, k1, ks2)
    for i in range(5):
        for r in _ROTS[i % 2]:
            x0 = x0 + x1
            x1 = (x1 << np.uint32(r)) | (x1 >> np.uint32(32 - r))
            x1 = x1 ^ x0
        x0 = x0 + ks[(i + 1) % 3]
        x1 = x1 + ks[(i + 2) % 3] + np.uint32(i + 1)
    return x0 ^ x1


def _u01(bits):
    fb = (bits >> np.uint32(9)) | np.uint32(0x3F800000)
    return jax.lax.bitcast_convert_type(fb, jnp.float32) - np.float32(1.0)


def _gumbel_from_u01(u01):
    u = jnp.maximum(_TINY, u01 + _TINY)
    return -jnp.log(-jnp.log(u))


def _normal_from_u01(u01):
    u = jnp.maximum(_NLO, u01 * _NSPAN + _NLO)
    return _SQRT2 * jax.lax.erf_inv(u)


def _bf(v):
    return v.astype(jnp.bfloat16).astype(jnp.float32)


def _onehot_argmax4(ys):
    m = jnp.maximum(jnp.maximum(ys[0], ys[1]), jnp.maximum(ys[2], ys[3]))
    cand = [jnp.where(ys[c] == m, np.int32(c), np.int32(4)) for c in range(4)]
    cmin = jnp.minimum(jnp.minimum(cand[0], cand[1]),
                       jnp.minimum(cand[2], cand[3]))
    return [(cand[c] == cmin).astype(jnp.float32) for c in range(4)]


_EDGE_I = 2
_EB = _EDGE_I * 768


def _edges_kernel(bp_ref, x_ref, o_ref):
    x = x_ref[...]
    bp = bp_ref[...]

    row = jax.lax.broadcasted_iota(jnp.int32, (_EB, 128), 0)
    lane = jax.lax.broadcasted_iota(jnp.int32, (_EB, 128), 1)
    j = row & 63
    rc = row >> 6
    iblk = (rc >= 12).astype(jnp.int32)
    ch = rc - 12 * iblk
    c4 = ch & 3
    rlog = (lane << 12) + (pl.program_id(0) * _EDGE_I + iblk) * 64 + j
    ig = (rlog << 2) | c4

    def sel(vals):
        v0, v1, v2 = (np.uint32(v) for v in vals)
        return jnp.where(ch < 4, v0, jnp.where(ch < 8, v1, v2)).astype(jnp.uint32)

    k0 = sel((_KA[0], _KB2[0], _KCON[0]))
    k1 = sel((_KA[1], _KB2[1], _KCON[1]))
    ks2 = k0 ^ k1 ^ np.uint32(0x1BD11BDA)

    gum = _gumbel_from_u01(_u01(_tf2x32(k0, k1, ks2, ig.astype(jnp.uint32))))

    xb = _bf(x)
    ob = _bf((np.float32(1.0) - bp) * np.float32(0.25))
    db = _bf(bp + (np.float32(1.0) - bp) * np.float32(0.25))

    for blk in range(_EDGE_I):
        for g in range(3):
            base = blk * 768 + g * 256
            xg = [xb[base + c * 64:base + (c + 1) * 64] for c in range(4)]
            ys = []
            for c in range(4):
                acc = None
                for jj in range(4):
                    t = xg[jj] * (db if jj == c else ob)
                    acc = t if acc is None else acc + t
                sl = slice(base + c * 64, base + (c + 1) * 64)
                ys.append(jnp.log(jnp.maximum(acc, _LOGEPS)) + gum[sl])
            oh = _onehot_argmax4(ys)
            for c in range(4):
                o_ref[base + c * 64:base + (c + 1) * 64, :] = oh[c]


def _nodes_kernel(x_ref, bp_ref, bv_ref, ob5_ref, db5_ref, on_ref, og_ref):
    x = x_ref[...]
    bp = bp_ref[...]
    bv = bv_ref[...]
    ob5 = ob5_ref[...]
    db5 = db5_ref[...]

    row = jax.lax.broadcasted_iota(jnp.int32, (896, 128), 0)
    lane = jax.lax.broadcasted_iota(jnp.int32, (896, 128), 1)
    n = row & 63
    ch = row >> 6
    rn = (lane << 6) + n

    is_flag = ch == 0
    is_cat = jnp.logical_and(ch >= 1, ch <= 5)
    is_g = jnp.logical_and(ch >= 6, ch <= 12)

    i1 = jnp.where(is_flag, 2 * rn,
                   jnp.where(is_cat, 5 * rn + (ch - 1),
                             jnp.where(is_g, 7 * rn + (ch - 6), 2 * rn + 1)))

    def sel(vals):
        v0, v1, v2, v3 = (np.uint32(v) for v in vals)
        return jnp.where(is_flag, v0,
                         jnp.where(is_cat, v1,
                                   jnp.where(is_g, v2, v3))).astype(jnp.uint32)

    k0 = sel((_KB[0], _KC[0], _KG[0], _KB[0]))
    k1 = sel((_KB[1], _KC[1], _KG[1], _KB[1]))
    ks2 = k0 ^ k1 ^ np.uint32(0x1BD11BDA)
    u = _u01(_tf2x32(k0, k1, ks2, i1.astype(jnp.uint32)))

    one = np.float32(1.0)

    f = x[0:64]
    gum0 = _gumbel_from_u01(u[0:64])
    gum2 = _gumbel_from_u01(u[832:896])
    ob2 = _bf((one - bp) * np.float32(0.5))
    db2 = _bf(bp + (one - bp) * np.float32(0.5))
    fb0 = _bf(one - f)
    fb1 = _bf(f)
    y0 = jnp.log(jnp.maximum(fb0 * db2 + fb1 * ob2, _LOGEPS)) + gum0
    y1 = jnp.log(jnp.maximum(fb0 * ob2 + fb1 * db2, _LOGEPS)) + gum2
    on_ref[0:64, :] = (y1 > y0).astype(jnp.float32)

    xc = [_bf(x[jj * 64:(jj + 1) * 64]) for jj in range(1, 6)]
    ys = []
    for c in range(5):
        acc = None
        for jj in range(5):
            t = xc[jj] * (db5 if jj == c else ob5)
            acc = t if acc is None else acc + t
        sl = slice((c + 1) * 64, (c + 2) * 64)
        ys.append(jnp.log(jnp.maximum(acc, _LOGEPS)) + _gumbel_from_u01(u[sl]))
    m = jnp.maximum(jnp.maximum(jnp.maximum(ys[0], ys[1]),
                                jnp.maximum(ys[2], ys[3])), ys[4])
    cand = [jnp.where(ys[c] == m, np.int32(c), np.int32(5)) for c in range(5)]
    cmin = jnp.minimum(jnp.minimum(jnp.minimum(cand[0], cand[1]),
                                   jnp.minimum(cand[2], cand[3])), cand[4])
    for c in range(5):
        on_ref[(c + 1) * 64:(c + 2) * 64, :] = (cand[c] == cmin).astype(jnp.float32)

    nrm = _normal_from_u01(u[384:832])
    on_ref[384:832, :] = bp * x[384:832] + bv * nrm
    og_ref[...] = nrm


def kernel(nodes, edges, timestep):
    sqrt_cum_prec, sqrt_cum_var = _schedule()
    bp = sqrt_cum_prec[timestep][None, :]
    bv = sqrt_cum_var[timestep][None, :]
    ob5 = ((1.0 - bp) / 5.0).astype(jnp.bfloat16).astype(jnp.float32)
    db5 = (bp + (1.0 - bp) / 5.0).astype(jnp.bfloat16).astype(jnp.float32)

    ev = edges.transpose(1, 3, 2, 0).reshape(49152, 128)
    noisy_edges = pl.pallas_call(
        _edges_kernel,
        grid=(64 // _EDGE_I,),
        in_specs=[
            pl.BlockSpec((1, 128), lambda i: (0, 0)),
            pl.BlockSpec((_EB, 128), lambda i: (i, 0)),
        ],
        out_specs=pl.BlockSpec((_EB, 128), lambda i: (i, 0)),
        out_shape=jax.ShapeDtypeStruct((49152, 128), jnp.float32),
    )(bp, ev).reshape(64, 12, 64, 128).transpose(3, 0, 2, 1)

    nv = nodes.transpose(2, 1, 0).reshape(832, 128)
    noisy_nodes, gnoise = pl.pallas_call(
        _nodes_kernel,
        grid=(1,),
        in_specs=[
            pl.BlockSpec((832, 128), lambda i: (0, 0)),
            pl.BlockSpec((1, 128), lambda i: (0, 0)),
            pl.BlockSpec((1, 128), lambda i: (0, 0)),
            pl.BlockSpec((1, 128), lambda i: (0, 0)),
            pl.BlockSpec((1, 128), lambda i: (0, 0)),
        ],
        out_specs=[
            pl.BlockSpec((832, 128), lambda i: (0, 0)),
            pl.BlockSpec((448, 128), lambda i: (0, 0)),
        ],
        out_shape=[
            jax.ShapeDtypeStruct((832, 128), jnp.float32),
            jax.ShapeDtypeStruct((448, 128), jnp.float32),
        ],
    )(nv, bp, bv, ob5, db5)

    return (noisy_nodes.reshape(13, 64, 128).transpose(2, 1, 0),
            noisy_edges,
            gnoise.reshape(7, 64, 128).transpose(2, 1, 0))
